# initial kernel scaffold (unmeasured)
import jax
import jax.numpy as jnp
from jax import lax
from jax.experimental import pallas as pl
from jax.experimental.pallas import tpu as pltpu

N_DEV = 4
E_LOC = 4
E_GLB = N_DEV * E_LOC
CAP = 256
D = 1024
F = 2048


def _moe_body(xb_ref, w1_ref, w2_ref, out_ref, recv_ref, res_ref,
              send1, recv1, send2, recv2):
    my_x = lax.axis_index("x")
    my_y = lax.axis_index("y")
    my_z = lax.axis_index("z")

    barrier = pltpu.get_barrier_semaphore()
    for k in range(1, N_DEV):
        peer = (my_y + k) % N_DEV
        pl.semaphore_signal(barrier, inc=1,
                            device_id=(my_x, peer, my_z),
                            device_id_type=pl.DeviceIdType.MESH)
    pl.semaphore_wait(barrier, N_DEV - 1)

    sends1 = []
    for k in range(N_DEV):
        dst = (my_y + k) % N_DEV
        rdma = pltpu.make_async_remote_copy(
            src_ref=xb_ref.at[dst],
            dst_ref=recv_ref.at[my_y],
            send_sem=send1.at[k],
            recv_sem=recv1.at[my_y],
            device_id=(my_x, dst, my_z),
            device_id_type=pl.DeviceIdType.MESH,
        )
        rdma.start()
        sends1.append(rdma)

    for k in range(N_DEV):
        src = (my_y + k) % N_DEV
        rdma = pltpu.make_async_remote_copy(
            src_ref=xb_ref.at[src],
            dst_ref=recv_ref.at[src],
            send_sem=send1.at[k],
            recv_sem=recv1.at[src],
            device_id=(my_x, my_y, my_z),
            device_id_type=pl.DeviceIdType.MESH,
        )
        rdma.wait_recv()
    for rdma in sends1:
        rdma.wait_send()

    for le in range(E_LOC):
        toks = recv_ref[:, le].reshape(N_DEV * CAP, D)
        h = jnp.maximum(
            jnp.dot(toks, w1_ref[le], preferred_element_type=jnp.float32),
            0.0,
        ).astype(jnp.bfloat16)
        y = jnp.dot(h, w2_ref[le], preferred_element_type=jnp.float32)
        res_ref[:, le] = y.astype(jnp.bfloat16).reshape(N_DEV, CAP, D)

    sends2 = []
    for k in range(N_DEV):
        dst = (my_y + k) % N_DEV
        rdma = pltpu.make_async_remote_copy(
            src_ref=res_ref.at[dst],
            dst_ref=out_ref.at[my_y],
            send_sem=send2.at[k],
            recv_sem=recv2.at[my_y],
            device_id=(my_x, dst, my_z),
            device_id_type=pl.DeviceIdType.MESH,
        )
        rdma.start()
        sends2.append(rdma)

    for k in range(N_DEV):
        src = (my_y + k) % N_DEV
        rdma = pltpu.make_async_remote_copy(
            src_ref=res_ref.at[src],
            dst_ref=out_ref.at[src],
            send_sem=send2.at[k],
            recv_sem=recv2.at[src],
            device_id=(my_x, my_y, my_z),
            device_id_type=pl.DeviceIdType.MESH,
        )
        rdma.wait_recv()
    for rdma in sends2:
        rdma.wait_send()


def _moe_a2a(xb, w1, w2):
    return pl.pallas_call(
        _moe_body,
        out_shape=jax.ShapeDtypeStruct((N_DEV, E_LOC, CAP, D), jnp.bfloat16),
        in_specs=[pl.BlockSpec(memory_space=pltpu.VMEM)] * 3,
        out_specs=pl.BlockSpec(memory_space=pltpu.VMEM),
        scratch_shapes=[
            pltpu.VMEM((N_DEV, E_LOC, CAP, D), jnp.bfloat16),
            pltpu.VMEM((N_DEV, E_LOC, CAP, D), jnp.bfloat16),
            pltpu.SemaphoreType.DMA((N_DEV,)),
            pltpu.SemaphoreType.DMA((N_DEV,)),
            pltpu.SemaphoreType.DMA((N_DEV,)),
            pltpu.SemaphoreType.DMA((N_DEV,)),
        ],
        compiler_params=pltpu.CompilerParams(collective_id=0),
    )(xb, w1, w2)


def kernel(x, assign, W1, W2):
    t = x.shape[0]

    oh = (assign[:, None] == jnp.arange(E_GLB)[None, :]).astype(jnp.int32)
    rank = jnp.take_along_axis(
        jnp.cumsum(oh, axis=0), assign[:, None], axis=1
    )[:, 0] - 1
    slot = assign * CAP + rank

    xflat = jnp.zeros((E_GLB * CAP, D), jnp.bfloat16)
    xflat = xflat.at[slot].set(x.astype(jnp.bfloat16), mode="drop")
    xb = xflat.reshape(N_DEV, E_LOC, CAP, D)

    yb = _moe_a2a(xb, W1.astype(jnp.bfloat16), W2.astype(jnp.bfloat16))

    out = yb.reshape(E_GLB * CAP, D)[slot]
    return out.astype(jnp.float32)


# baseline (device time: 260429 ns/iter reference)
import jax
import jax.numpy as jnp
from jax import lax
from jax.experimental import pallas as pl
from jax.experimental.pallas import tpu as pltpu

N_DEV = 4
E_LOC = 4
E_GLB = N_DEV * E_LOC
CAP = 192
D = 1024
F = 2048


def _moe_body(xb_ref, w1_ref, w2_ref, out_ref, recv_ref,
              send1, recv1, send2, recv2):
    my_x = lax.axis_index("x")
    my_y = lax.axis_index("y")
    my_z = lax.axis_index("z")

    barrier = pltpu.get_barrier_semaphore()
    for k in range(1, N_DEV):
        peer = (my_y + k) % N_DEV
        pl.semaphore_signal(barrier, inc=1,
                            device_id=(my_x, peer, my_z),
                            device_id_type=pl.DeviceIdType.MESH)
    pl.semaphore_wait(barrier, N_DEV - 1)

    sends1 = []
    for k in range(N_DEV):
        dst = (my_y + k) % N_DEV
        rdma = pltpu.make_async_remote_copy(
            src_ref=xb_ref.at[dst],
            dst_ref=recv_ref.at[my_y],
            send_sem=send1.at[k],
            recv_sem=recv1.at[my_y],
            device_id=(my_x, dst, my_z),
            device_id_type=pl.DeviceIdType.MESH,
        )
        rdma.start()
        sends1.append(rdma)

    for k in range(N_DEV):
        src = (my_y + k) % N_DEV
        rdma = pltpu.make_async_remote_copy(
            src_ref=xb_ref.at[src],
            dst_ref=recv_ref.at[src],
            send_sem=send1.at[k],
            recv_sem=recv1.at[src],
            device_id=(my_x, my_y, my_z),
            device_id_type=pl.DeviceIdType.MESH,
        )
        rdma.wait_recv()
    for rdma in sends1:
        rdma.wait_send()

    for le in range(E_LOC):
        toks = recv_ref[:, le].reshape(N_DEV * CAP, D)
        h = jnp.maximum(
            jnp.dot(toks, w1_ref[le], preferred_element_type=jnp.float32),
            0.0,
        ).astype(jnp.bfloat16)
        y = jnp.dot(h, w2_ref[le], preferred_element_type=jnp.float32)
        recv_ref[:, le] = y.astype(jnp.bfloat16).reshape(N_DEV, CAP, D)

    sends2 = []
    for k in range(N_DEV):
        dst = (my_y + k) % N_DEV
        rdma = pltpu.make_async_remote_copy(
            src_ref=recv_ref.at[dst],
            dst_ref=out_ref.at[my_y],
            send_sem=send2.at[k],
            recv_sem=recv2.at[my_y],
            device_id=(my_x, dst, my_z),
            device_id_type=pl.DeviceIdType.MESH,
        )
        rdma.start()
        sends2.append(rdma)

    for k in range(N_DEV):
        src = (my_y + k) % N_DEV
        rdma = pltpu.make_async_remote_copy(
            src_ref=recv_ref.at[src],
            dst_ref=out_ref.at[src],
            send_sem=send2.at[k],
            recv_sem=recv2.at[src],
            device_id=(my_x, my_y, my_z),
            device_id_type=pl.DeviceIdType.MESH,
        )
        rdma.wait_recv()
    for rdma in sends2:
        rdma.wait_send()


def _moe_a2a(xb, w1, w2):
    return pl.pallas_call(
        _moe_body,
        out_shape=jax.ShapeDtypeStruct((N_DEV, E_LOC, CAP, D), jnp.bfloat16),
        in_specs=[pl.BlockSpec(memory_space=pltpu.VMEM)] * 3,
        out_specs=pl.BlockSpec(memory_space=pltpu.VMEM),
        scratch_shapes=[
            pltpu.VMEM((N_DEV, E_LOC, CAP, D), jnp.bfloat16),
            pltpu.SemaphoreType.DMA((N_DEV,)),
            pltpu.SemaphoreType.DMA((N_DEV,)),
            pltpu.SemaphoreType.DMA((N_DEV,)),
            pltpu.SemaphoreType.DMA((N_DEV,)),
        ],
        compiler_params=pltpu.CompilerParams(
            collective_id=0, vmem_limit_bytes=100 * 1024 * 1024
        ),
    )(xb, w1, w2)


def kernel(x, assign, W1, W2):
    t = x.shape[0]

    oh = (assign[:, None] == jnp.arange(E_GLB)[None, :]).astype(jnp.int32)
    rank = jnp.take_along_axis(
        jnp.cumsum(oh, axis=0), assign[:, None], axis=1
    )[:, 0] - 1
    slot = assign * CAP + rank

    xflat = jnp.zeros((E_GLB * CAP, D), jnp.bfloat16)
    xflat = xflat.at[slot].set(x.astype(jnp.bfloat16), mode="drop")
    xb = xflat.reshape(N_DEV, E_LOC, CAP, D)

    yb = _moe_a2a(xb, W1.astype(jnp.bfloat16), W2.astype(jnp.bfloat16))

    out = yb.reshape(E_GLB * CAP, D)[slot]
    return out.astype(jnp.float32)


# device time: 259839 ns/iter; 1.0023x vs baseline; 1.0023x over previous
import jax
import jax.numpy as jnp
from jax import lax
from jax.experimental import pallas as pl
from jax.experimental.pallas import tpu as pltpu

N_DEV = 4
E_LOC = 4
E_GLB = N_DEV * E_LOC
CAP = 192
D = 1024
F = 2048


def _cvt_body(src_ref, dst_ref):
    dst_ref[...] = src_ref[...].astype(jnp.bfloat16)


def _to_bf16(w):
    blk = (1,) + w.shape[1:]
    return pl.pallas_call(
        _cvt_body,
        grid=(w.shape[0],),
        in_specs=[pl.BlockSpec(blk, lambda i: (i, 0, 0))],
        out_specs=pl.BlockSpec(blk, lambda i: (i, 0, 0)),
        out_shape=jax.ShapeDtypeStruct(w.shape, jnp.bfloat16),
    )(w)


def _moe_body(xb_ref, w1_ref, w2_ref, out_ref, recv_ref,
              send1, recv1, send2, recv2):
    my_x = lax.axis_index("x")
    my_y = lax.axis_index("y")
    my_z = lax.axis_index("z")

    barrier = pltpu.get_barrier_semaphore()
    for k in range(1, N_DEV):
        peer = (my_y + k) % N_DEV
        pl.semaphore_signal(barrier, inc=1,
                            device_id=(my_x, peer, my_z),
                            device_id_type=pl.DeviceIdType.MESH)
    pl.semaphore_wait(barrier, N_DEV - 1)

    sends1 = []
    for k in range(N_DEV):
        dst = (my_y + k) % N_DEV
        rdma = pltpu.make_async_remote_copy(
            src_ref=xb_ref.at[dst],
            dst_ref=recv_ref.at[my_y],
            send_sem=send1.at[k],
            recv_sem=recv1.at[my_y],
            device_id=(my_x, dst, my_z),
            device_id_type=pl.DeviceIdType.MESH,
        )
        rdma.start()
        sends1.append(rdma)

    for k in range(N_DEV):
        src = (my_y + k) % N_DEV
        rdma = pltpu.make_async_remote_copy(
            src_ref=xb_ref.at[src],
            dst_ref=recv_ref.at[src],
            send_sem=send1.at[k],
            recv_sem=recv1.at[src],
            device_id=(my_x, my_y, my_z),
            device_id_type=pl.DeviceIdType.MESH,
        )
        rdma.wait_recv()
    for rdma in sends1:
        rdma.wait_send()

    for le in range(E_LOC):
        toks = recv_ref[:, le].reshape(N_DEV * CAP, D)
        h = jnp.maximum(
            jnp.dot(toks, w1_ref[le], preferred_element_type=jnp.float32),
            0.0,
        ).astype(jnp.bfloat16)
        y = jnp.dot(h, w2_ref[le], preferred_element_type=jnp.float32)
        recv_ref[:, le] = y.astype(jnp.bfloat16).reshape(N_DEV, CAP, D)

    sends2 = []
    for k in range(N_DEV):
        dst = (my_y + k) % N_DEV
        rdma = pltpu.make_async_remote_copy(
            src_ref=recv_ref.at[dst],
            dst_ref=out_ref.at[my_y],
            send_sem=send2.at[k],
            recv_sem=recv2.at[my_y],
            device_id=(my_x, dst, my_z),
            device_id_type=pl.DeviceIdType.MESH,
        )
        rdma.start()
        sends2.append(rdma)

    for k in range(N_DEV):
        src = (my_y + k) % N_DEV
        rdma = pltpu.make_async_remote_copy(
            src_ref=recv_ref.at[src],
            dst_ref=out_ref.at[src],
            send_sem=send2.at[k],
            recv_sem=recv2.at[src],
            device_id=(my_x, my_y, my_z),
            device_id_type=pl.DeviceIdType.MESH,
        )
        rdma.wait_recv()
    for rdma in sends2:
        rdma.wait_send()


def _moe_a2a(xb, w1, w2):
    return pl.pallas_call(
        _moe_body,
        out_shape=jax.ShapeDtypeStruct((N_DEV, E_LOC, CAP, D), jnp.bfloat16),
        in_specs=[pl.BlockSpec(memory_space=pltpu.VMEM)] * 3,
        out_specs=pl.BlockSpec(memory_space=pltpu.VMEM),
        scratch_shapes=[
            pltpu.VMEM((N_DEV, E_LOC, CAP, D), jnp.bfloat16),
            pltpu.SemaphoreType.DMA((N_DEV,)),
            pltpu.SemaphoreType.DMA((N_DEV,)),
            pltpu.SemaphoreType.DMA((N_DEV,)),
            pltpu.SemaphoreType.DMA((N_DEV,)),
        ],
        compiler_params=pltpu.CompilerParams(
            collective_id=0, vmem_limit_bytes=100 * 1024 * 1024
        ),
    )(xb, w1, w2)


def kernel(x, assign, W1, W2):
    t = x.shape[0]

    oh = (assign[:, None] == jnp.arange(E_GLB)[None, :]).astype(jnp.int32)
    rank = jnp.take_along_axis(
        jnp.cumsum(oh, axis=0), assign[:, None], axis=1
    )[:, 0] - 1
    slot = assign * CAP + rank

    xflat = jnp.zeros((E_GLB * CAP, D), jnp.bfloat16)
    xflat = xflat.at[slot].set(x.astype(jnp.bfloat16), mode="drop")
    xb = xflat.reshape(N_DEV, E_LOC, CAP, D)

    yb = _moe_a2a(xb, _to_bf16(W1), _to_bf16(W2))

    out = yb.reshape(E_GLB * CAP, D)[slot]
    return out.astype(jnp.float32)
